# baseline (speedup 1.0000x reference)
import jax
import jax.numpy as jnp
from jax import lax
from jax.experimental import pallas as pl
from jax.experimental.pallas import tpu as pltpu
from jax.experimental.pallas import tpu_sc as plsc

NC = 2
NS = 16
NW = NC * NS
L = 16

D = 64
T = 16384
SEG = 16
SEGLEN = 1024
V = 1000000

CH = 384
NBUF = 4
TAIL_K = V // CH
TAIL_W = V - TAIL_K * CH
CH_PER_W = TAIL_K // NW + 1


def _body(textf_hbm, tabT_hbm, tail_hbm, out_hbm,
          locp_v, buf0_v, buf1_v, buf2_v, buf3_v, acc_v,
          scr_r, scr_s, row_v, red_v, shared_v, sem0, sem1, sem2, sem3):
    c = lax.axis_index("c")
    s = lax.axis_index("s")
    w = s * NC + c
    iota = lax.iota(jnp.int32, L)

    def div_ch(x):
        return ((x >> 7) * 21846) >> 16

    bufs = (buf0_v, buf1_v, buf2_v, buf3_v)
    sems = (sem0, sem1, sem2, sem3)

    def chunk_descr(idx, nb):
        k = w + NW * idx
        full = pltpu.make_async_copy(
            tabT_hbm.at[:, pl.ds(k * CH, CH)], bufs[nb], sems[nb])
        tail = pltpu.make_async_copy(
            tail_hbm, bufs[nb].at[:, pl.ds(0, 2 * D)], sems[nb])
        return k, full, tail

    def start_chunk(idx, nb):
        k, full, tail = chunk_descr(idx, nb)
        pl.when(k < TAIL_K)(full.start)
        pl.when(k == TAIL_K)(tail.start)

    def wait_chunk(idx, nb):
        k, full, tail = chunk_descr(idx, nb)
        pl.when(k < TAIL_K)(full.wait)
        pl.when(k == TAIL_K)(tail.wait)

    def process_chunk(idx, nb, cnt_):
        k = w + NW * idx
        buf = bufs[nb]

        @pl.when(k <= TAIL_K)
        def _():
            nstep = (cnt_ + (L - 1)) // L

            def per_vreg(m, _):
                pk = locp_v[pl.ds(m * L, L)]
                rv = pk & ((1 << 20) - 1)
                valid = (iota + m * L) < cnt_
                match = (div_ch(rv) == k) & valid
                nm = jnp.sum(match.astype(jnp.int32))

                @pl.when(nm > 0)
                def _():
                    plsc.store_compressed(scr_r.at[pl.ds(0, L)], rv,
                                          mask=match)
                    plsc.store_compressed(scr_s.at[pl.ds(0, L)], pk >> 20,
                                          mask=match)

                    def per_tok(t, __):
                        tvec = jnp.full((L,), 0, jnp.int32) + t
                        cidx = plsc.load_gather(scr_r, [tvec]) - k * CH
                        sgv = plsc.load_gather(scr_s, [tvec])
                        for g in range(D // L):
                            vals = plsc.load_gather(
                                buf, [iota + g * L, cidx])
                            plsc.addupdate_scatter(
                                acc_v, [sgv * D + g * L + iota], vals)
                        return 0

                    lax.fori_loop(0, nm, per_tok, 0)
                return 0

            lax.fori_loop(0, nstep, per_vreg, 0)

    for b in range(NBUF - 1):
        start_chunk(b, b)

    pltpu.sync_copy(textf_hbm, buf3_v.at[:, pl.ds(0, T // D)])

    def p0(i, cnt):
        rvf = buf3_v[i // L, pl.ds((i % L) * L, L)]
        rv = plsc.bitcast(rvf, jnp.int32)
        sv = (iota + i * L) >> 10
        mine = (div_ch(rv) & (NW - 1)) == w
        plsc.store_compressed(locp_v.at[pl.ds(cnt, L)], rv | (sv << 20),
                              mask=mine)
        return cnt + jnp.sum(mine.astype(jnp.int32))

    cnt = lax.fori_loop(0, T // L, p0, jnp.int32(0))

    zero16 = jnp.zeros((L,), jnp.float32)
    for r in range(SEG * D // L):
        acc_v[pl.ds(r * L, L)] = zero16

    def per_group(i, cnt_):
        for b in range(NBUF):
            idx = i * NBUF + b
            wait_chunk(idx, b)
            start_chunk(idx + (NBUF - 1), (b + NBUF - 1) % NBUF)
            process_chunk(idx, b, cnt_)
        return cnt_

    lax.fori_loop(0, CH_PER_W // NBUF + 1, per_group, cnt)

    pltpu.sync_copy(acc_v, shared_v.at[s])
    plsc.subcore_barrier()
    scale = jnp.float32(1.0 / SEGLEN)

    @pl.when(s < SEG // 2)
    def _():
        pltpu.sync_copy(shared_v.at[:, pl.ds(s * 2 * D, 2 * D)], red_v)
        for half in range(2):
            for g in range(D // L):
                acc = zero16
                for ws in range(NS):
                    acc = acc + red_v[ws, pl.ds(half * D + g * L, L)]
                row_v[pl.ds(g * L, L)] = acc * scale
            pltpu.sync_copy(row_v, out_hbm.at[c, s * 2 + half])


@jax.jit
def _pooled(text, table):
    tabT = jnp.transpose(table, (1, 0))
    textf = lax.bitcast_convert_type(text, jnp.float32).reshape(D, T // D)
    tail = jnp.zeros((2 * D, D), jnp.float32).at[:TAIL_W, :].set(
        table[TAIL_K * CH:, :]).T
    kfn = pl.kernel(
        _body,
        out_type=jax.ShapeDtypeStruct((NC, SEG, D), jnp.float32),
        mesh=plsc.VectorSubcoreMesh(
            core_axis_name="c", subcore_axis_name="s",
            num_cores=NC, num_subcores=NS,
        ),
        scratch_types=[
            pltpu.VMEM((T + L,), jnp.int32),
            pltpu.VMEM((D, CH), jnp.float32),
            pltpu.VMEM((D, CH), jnp.float32),
            pltpu.VMEM((D, CH), jnp.float32),
            pltpu.VMEM((D, CH), jnp.float32),
            pltpu.VMEM((SEG * D,), jnp.float32),
            pltpu.VMEM((L,), jnp.int32),
            pltpu.VMEM((L,), jnp.int32),
            pltpu.VMEM((D,), jnp.float32),
            pltpu.VMEM((NS, 2 * D), jnp.float32),
            pltpu.VMEM_SHARED((NS, SEG * D), jnp.float32),
            pltpu.SemaphoreType.DMA,
            pltpu.SemaphoreType.DMA,
            pltpu.SemaphoreType.DMA,
            pltpu.SemaphoreType.DMA,
        ],
        compiler_params=pltpu.CompilerParams(use_tc_tiling_on_sc=True, needs_layout_passes=False),
    )
    return kfn(textf, tabT, tail)


def kernel(text, emb_table):
    parts = _pooled(text, emb_table)
    pooled = parts[0] + parts[1]
    return jnp.transpose(pooled, (1, 0))[None]


# speedup vs baseline: 1.2161x; 1.2161x over previous
import jax
import jax.numpy as jnp
from jax import lax
from jax.experimental import pallas as pl
from jax.experimental.pallas import tpu as pltpu
from jax.experimental.pallas import tpu_sc as plsc

NC = 2
NS = 16
NW = NC * NS
L = 16

D = 64
T = 16384
SEG = 16
SEGLEN = 1024
V = 1000000

CH = 512
CH_SH = 9
NBUF = 3
TAIL_K = V // CH
TAIL_W = V - TAIL_K * CH
CH_PER_W = TAIL_K // NW + 1


def _body(textf_hbm, tabT_hbm, tail_hbm, out_hbm,
          locp_v, buf0_v, buf1_v, buf2_v, acc_v,
          scr_r, scr_s, row_v, red_v, shared_v, sem0, sem1, sem2):
    c = lax.axis_index("c")
    s = lax.axis_index("s")
    w = s * NC + c
    iota = lax.iota(jnp.int32, L)

    bufs = (buf0_v, buf1_v, buf2_v)
    sems = (sem0, sem1, sem2)

    def chunk_descr(idx, nb):
        k = w + NW * idx
        full = pltpu.make_async_copy(
            tabT_hbm.at[:, pl.ds(k * CH, CH)], bufs[nb], sems[nb])
        tail = pltpu.make_async_copy(
            tail_hbm, bufs[nb].at[:, pl.ds(0, 2 * D)], sems[nb])
        return k, full, tail

    def start_chunk(idx, nb):
        k, full, tail = chunk_descr(idx, nb)
        pl.when(k < TAIL_K)(full.start)
        pl.when(k == TAIL_K)(tail.start)

    def wait_chunk(idx, nb):
        k, full, tail = chunk_descr(idx, nb)
        pl.when(k < TAIL_K)(full.wait)
        pl.when(k == TAIL_K)(tail.wait)

    def process_chunk(idx, nb, cnt_):
        k = w + NW * idx
        buf = bufs[nb]

        @pl.when(k <= TAIL_K)
        def _():
            nstep = (cnt_ + (L - 1)) // L

            def per_vreg(m, _):
                pk = locp_v[pl.ds(m * L, L)]
                rv = pk & ((1 << 20) - 1)
                valid = (iota + m * L) < cnt_
                match = ((rv >> CH_SH) == k) & valid
                nm = jnp.sum(match.astype(jnp.int32))

                @pl.when(nm > 0)
                def _():
                    plsc.store_compressed(scr_r.at[pl.ds(0, L)], rv,
                                          mask=match)
                    plsc.store_compressed(scr_s.at[pl.ds(0, L)], pk >> 20,
                                          mask=match)

                    def per_tok(t, __):
                        tvec = jnp.full((L,), 0, jnp.int32) + t
                        cidx = plsc.load_gather(scr_r, [tvec]) & (CH - 1)
                        sgv = plsc.load_gather(scr_s, [tvec])
                        for g in range(D // L):
                            vals = plsc.load_gather(
                                buf, [iota + g * L, cidx])
                            plsc.addupdate_scatter(
                                acc_v, [sgv * D + g * L + iota], vals)
                        return 0

                    lax.fori_loop(0, nm, per_tok, 0)
                return 0

            lax.fori_loop(0, nstep, per_vreg, 0)

    for b in range(NBUF - 1):
        start_chunk(b, b)

    pltpu.sync_copy(textf_hbm, buf2_v.at[:, pl.ds(0, T // D)])

    def p0(i, cnt):
        rvf = buf2_v[i // L, pl.ds((i % L) * L, L)]
        rv = plsc.bitcast(rvf, jnp.int32)
        sv = (iota + i * L) >> 10
        mine = ((rv >> CH_SH) & (NW - 1)) == w
        plsc.store_compressed(locp_v.at[pl.ds(cnt, L)], rv | (sv << 20),
                              mask=mine)
        return cnt + jnp.sum(mine.astype(jnp.int32))

    cnt = lax.fori_loop(0, T // L, p0, jnp.int32(0))

    zero16 = jnp.zeros((L,), jnp.float32)
    for r in range(SEG * D // L):
        acc_v[pl.ds(r * L, L)] = zero16

    def per_group(i, cnt_):
        for b in range(NBUF):
            idx = i * NBUF + b
            wait_chunk(idx, b)
            start_chunk(idx + (NBUF - 1), (b + NBUF - 1) % NBUF)
            process_chunk(idx, b, cnt_)
        return cnt_

    lax.fori_loop(0, CH_PER_W // NBUF + 1, per_group, cnt)

    pltpu.sync_copy(acc_v, shared_v.at[s])
    plsc.subcore_barrier()
    scale = jnp.float32(1.0 / SEGLEN)

    @pl.when(s < SEG // 2)
    def _():
        pltpu.sync_copy(shared_v.at[:, pl.ds(s * 2 * D, 2 * D)], red_v)
        for half in range(2):
            for g in range(D // L):
                acc = zero16
                for ws in range(NS):
                    acc = acc + red_v[ws, pl.ds(half * D + g * L, L)]
                row_v[pl.ds(g * L, L)] = acc * scale
            pltpu.sync_copy(row_v, out_hbm.at[c, s * 2 + half])


@jax.jit
def _pooled(text, table):
    tabT = jnp.transpose(table, (1, 0))
    textf = lax.bitcast_convert_type(text, jnp.float32).reshape(D, T // D)
    tail = jnp.zeros((2 * D, D), jnp.float32).at[:TAIL_W, :].set(
        table[TAIL_K * CH:, :]).T
    kfn = pl.kernel(
        _body,
        out_type=jax.ShapeDtypeStruct((NC, SEG, D), jnp.float32),
        mesh=plsc.VectorSubcoreMesh(
            core_axis_name="c", subcore_axis_name="s",
            num_cores=NC, num_subcores=NS,
        ),
        scratch_types=[
            pltpu.VMEM((T + L,), jnp.int32),
            pltpu.VMEM((D, CH), jnp.float32),
            pltpu.VMEM((D, CH), jnp.float32),
            pltpu.VMEM((D, CH), jnp.float32),
            pltpu.VMEM((SEG * D,), jnp.float32),
            pltpu.VMEM((L,), jnp.int32),
            pltpu.VMEM((L,), jnp.int32),
            pltpu.VMEM((D,), jnp.float32),
            pltpu.VMEM((NS, 2 * D), jnp.float32),
            pltpu.VMEM_SHARED((NS, SEG * D), jnp.float32),
            pltpu.SemaphoreType.DMA,
            pltpu.SemaphoreType.DMA,
            pltpu.SemaphoreType.DMA,
        ],
        compiler_params=pltpu.CompilerParams(use_tc_tiling_on_sc=True, needs_layout_passes=False),
    )
    return kfn(textf, tabT, tail)


def kernel(text, emb_table):
    parts = _pooled(text, emb_table)
    pooled = parts[0] + parts[1]
    return jnp.transpose(pooled, (1, 0))[None]
